# Initial kernel scaffold; baseline (speedup 1.0000x reference)
import jax
import jax.numpy as jnp
from jax.experimental import pallas as pl


def kernel(x_compact, seqlens_tokens, W_qkv, W_out):
    raise NotImplementedError("write your pallas kernel here")


# speedup vs baseline: 2.2700x; 2.2700x over previous
import functools

import jax
import jax.numpy as jnp
import numpy as np
from jax import lax
from jax.experimental import pallas as pl
from jax.experimental.pallas import tpu as pltpu
from jax.experimental.pallas import tpu_sc as plsc

D_MODEL = 1024
N_HEADS = 16
HEAD_DIM = 64
TTOT = 8192
S = 8
SEG_LEN = 1024
W_EFF = 384
K_S = 3
N_CL = S * K_S
NPACK = N_CL * W_EFF
SEED_ROWS = (0, 4096, 8191)


def _qkv_r_body(x_ref, w_ref, qkv_ref, r_ref):
    xb = x_ref[...]
    qkv = jnp.dot(xb, w_ref[...], preferred_element_type=jnp.float32)
    qkv_ref[...] = qkv
    qm = qkv[:, 0:HEAD_DIM]
    km = qkv[:, D_MODEL:D_MODEL + HEAD_DIM]
    for h in range(1, N_HEADS):
        qm = qm + qkv[:, h * HEAD_DIM:(h + 1) * HEAD_DIM]
        km = km + qkv[:, D_MODEL + h * HEAD_DIM:D_MODEL + (h + 1) * HEAD_DIM]
    qm = qm * (1.0 / N_HEADS)
    km = km * (1.0 / N_HEADS)

    def _ln(t):
        m = jnp.mean(t, axis=-1, keepdims=True)
        v = jnp.mean((t - m) ** 2, axis=-1, keepdims=True)
        return (t - m) / jnp.sqrt(v + 1e-5)

    r = 0.5 * (_ln(qm) + _ln(km))
    nrm = jnp.sqrt(jnp.sum(r * r, axis=-1, keepdims=True))
    r_ref[...] = r / (nrm + 1e-6)


def _qkv_r(x, w_t):
    blk = 256
    return pl.pallas_call(
        _qkv_r_body,
        grid=(TTOT // blk,),
        in_specs=[
            pl.BlockSpec((blk, D_MODEL), lambda i: (i, 0)),
            pl.BlockSpec((D_MODEL, 3 * D_MODEL), lambda i: (0, 0)),
        ],
        out_specs=[
            pl.BlockSpec((blk, 3 * D_MODEL), lambda i: (i, 0)),
            pl.BlockSpec((blk, HEAD_DIM), lambda i: (i, 0)),
        ],
        out_shape=[
            jax.ShapeDtypeStruct((TTOT, 3 * D_MODEL), jnp.float32),
            jax.ShapeDtypeStruct((TTOT, HEAD_DIM), jnp.float32),
        ],
    )(x, w_t)


def _topk_body(r_ref, og_ref, ol_ref):
    seg = pl.program_id(0)
    rows = [r_ref[sr:sr + 1, :] for sr in SEED_ROWS]
    cmat = jnp.concatenate(rows, axis=0)
    cn = jnp.sqrt(jnp.sum(cmat * cmat, axis=-1, keepdims=True))
    cmat = cmat / (cn + 1e-6)
    r_seg = r_ref[pl.ds(seg * SEG_LEN, SEG_LEN), :]
    sims = lax.dot_general(cmat, r_seg, (((1,), (1,)), ((), ())),
                           preferred_element_type=jnp.float32)

    ri = lax.broadcasted_iota(jnp.int32, (SEG_LEN, SEG_LEN), 0)
    ci = lax.broadcasted_iota(jnp.int32, (SEG_LEN, SEG_LEN), 1)
    tri = (ri <= ci).astype(jnp.float32)
    tok_row = lax.broadcasted_iota(jnp.int32, (1, SEG_LEN), 1).astype(jnp.float32)
    slot_col = lax.broadcasted_iota(jnp.int32, (W_EFF, 1), 0)

    for c in range(K_S):
        sv = sims[c:c + 1, :]
        b = lax.bitcast_convert_type(sv, jnp.int32)
        key = jnp.where(b < 0, b ^ jnp.int32(0x7FFFFFFF), b)

        def bit_step(i, cur):
            bit = 31 - i
            inc = jnp.left_shift(jnp.int32(1), bit)
            cand = cur + inc
            cnt = jnp.sum((key >= cand).astype(jnp.int32))
            return jnp.where(cnt >= W_EFF, cand, cur)

        t_val = lax.fori_loop(0, 32, bit_step, jnp.int32(-2147483648))

        gt = key > t_val
        eq = key == t_val
        need = W_EFF - jnp.sum(gt.astype(jnp.int32))
        cum_eq = jnp.dot(eq.astype(jnp.float32), tri,
                         preferred_element_type=jnp.float32,
                         precision=lax.Precision.HIGHEST)
        sel = gt | (eq & (cum_eq <= need.astype(jnp.float32) + 0.5))
        self_f = sel.astype(jnp.float32)
        pos = jnp.dot(self_f, tri, preferred_element_type=jnp.float32,
                      precision=lax.Precision.HIGHEST) - 1.0
        p2 = ((slot_col.astype(jnp.float32) == pos) & sel).astype(jnp.float32)
        idx_f = lax.dot_general(tok_row * self_f, p2, (((1,), (1,)), ((), ())),
                                preferred_element_type=jnp.float32,
                                precision=lax.Precision.HIGHEST)
        idx_i = idx_f.astype(jnp.int32)
        ol_ref[0, c, :] = idx_i[0, :]
        og_ref[0, c, :] = idx_i[0, :] + seg * SEG_LEN


def _topk(r):
    return pl.pallas_call(
        _topk_body,
        grid=(S,),
        in_specs=[pl.BlockSpec((TTOT, HEAD_DIM), lambda s: (0, 0))],
        out_specs=[
            pl.BlockSpec((1, K_S, W_EFF), lambda s: (s, 0, 0)),
            pl.BlockSpec((1, K_S, W_EFF), lambda s: (s, 0, 0)),
        ],
        out_shape=[
            jax.ShapeDtypeStruct((S, K_S, W_EFF), jnp.int32),
            jax.ShapeDtypeStruct((S, K_S, W_EFF), jnp.int32),
        ],
    )(r)


def _gather_sc(qkv, order_flat):
    info = plsc.get_sparse_core_info()
    nw = info.num_cores * info.num_subcores
    rows_per_w = NPACK // nw
    chunk = 16
    n_chunks = rows_per_w // chunk
    mesh = plsc.VectorSubcoreMesh(core_axis_name="c", subcore_axis_name="s")

    @functools.partial(
        pl.kernel,
        out_type=jax.ShapeDtypeStruct((NPACK, 3 * D_MODEL), jnp.float32),
        mesh=mesh,
        scratch_types=[
            pltpu.VMEM((chunk,), jnp.int32),
            pltpu.VMEM((chunk, 3 * D_MODEL), jnp.float32),
            pltpu.SemaphoreType.DMA,
        ],
    )
    def k(table_hbm, idx_hbm, out_hbm, idx_v, rows_v, sem):
        wid = lax.axis_index("s") * info.num_cores + lax.axis_index("c")
        base = wid * rows_per_w

        def body(j, _):
            off = base + j * chunk
            pltpu.sync_copy(idx_hbm.at[pl.ds(off, chunk)], idx_v)
            pltpu.async_copy(table_hbm.at[idx_v], rows_v, sem).wait()
            pltpu.sync_copy(rows_v, out_hbm.at[pl.ds(off, chunk)])
            return 0

        lax.fori_loop(0, n_chunks, body, 0)

    return k(qkv, order_flat)


def _attn_body(pk_ref, out_ref):
    blk = pk_ref[...]
    scale = 1.0 / np.sqrt(HEAD_DIM)
    for h in range(N_HEADS):
        q = blk[:, h * HEAD_DIM:(h + 1) * HEAD_DIM]
        k = blk[:, D_MODEL + h * HEAD_DIM:D_MODEL + (h + 1) * HEAD_DIM]
        v = blk[:, 2 * D_MODEL + h * HEAD_DIM:2 * D_MODEL + (h + 1) * HEAD_DIM]
        s = lax.dot_general(q, k, (((1,), (1,)), ((), ())),
                            preferred_element_type=jnp.float32) * scale
        m = jnp.max(s, axis=-1, keepdims=True)
        p = jnp.exp(s - m)
        l = jnp.sum(p, axis=-1, keepdims=True)
        o = jnp.dot(p / l, v, preferred_element_type=jnp.float32)
        out_ref[:, h * HEAD_DIM:(h + 1) * HEAD_DIM] = o


def _attn(packed):
    return pl.pallas_call(
        _attn_body,
        grid=(N_CL,),
        in_specs=[pl.BlockSpec((W_EFF, 3 * D_MODEL), lambda i: (i, 0))],
        out_specs=pl.BlockSpec((W_EFF, D_MODEL), lambda i: (i, 0)),
        out_shape=jax.ShapeDtypeStruct((NPACK, D_MODEL), jnp.float32),
    )(packed)


def _scatter_sc(out_p, order_local_flat):
    info = plsc.get_sparse_core_info()
    seg_rows_packed = K_S * W_EFF
    segs_per_core = S // info.num_cores
    cols = 128
    half = SEG_LEN // 2
    mesh = plsc.VectorSubcoreMesh(core_axis_name="c", subcore_axis_name="s")

    @functools.partial(
        pl.kernel,
        out_type=jax.ShapeDtypeStruct((TTOT, D_MODEL), jnp.float32),
        mesh=mesh,
        scratch_types=[
            pltpu.VMEM((16, cols), jnp.float32),
            pltpu.VMEM((16,), jnp.int32),
            pltpu.VMEM((half, cols), jnp.float32),
        ],
    )
    def k(src_hbm, idx_hbm, out_hbm, rows_v, idx_v, acc):
        cid = lax.axis_index("c")
        sid = lax.axis_index("s")
        colb = sid % 8
        rhalf = sid // 8
        rbase = rhalf * half
        zeros16 = jnp.zeros((16,), jnp.float32)

        def per_seg(i, _):
            seg = cid * segs_per_core + i

            def zrow(rr, _):
                for j in range(cols // 16):
                    acc[rr, pl.ds(j * 16, 16)] = zeros16
                return 0

            lax.fori_loop(0, half, zrow, 0)
            base = seg * seg_rows_packed

            def group(g, _):
                off = base + g * 16
                pltpu.sync_copy(idx_hbm.at[pl.ds(off, 16)], idx_v)
                pltpu.sync_copy(
                    src_hbm.at[pl.ds(off, 16), pl.ds(colb * cols, cols)], rows_v)
                rv = idx_v[...]
                for i in range(16):
                    r = rv[i] - rbase

                    @pl.when((r >= 0) & (r < half))
                    def _():
                        for u in range(cols // 16):
                            plsc.addupdate(acc.at[r, pl.ds(u * 16, 16)],
                                           rows_v[i, pl.ds(u * 16, 16)])
                return 0

            lax.fori_loop(0, seg_rows_packed // 16, group, 0)
            pltpu.sync_copy(
                acc,
                out_hbm.at[pl.ds(seg * SEG_LEN + rbase, half),
                           pl.ds(colb * cols, cols)])
            return 0

        lax.fori_loop(0, segs_per_core, per_seg, 0)

    return k(out_p, order_local_flat)


def _proj_body(x_ref, w_ref, o_ref):
    o_ref[...] = jnp.dot(x_ref[...], w_ref[...],
                         preferred_element_type=jnp.float32)


def _proj(x, w_t):
    blk = 512
    return pl.pallas_call(
        _proj_body,
        grid=(TTOT // blk,),
        in_specs=[
            pl.BlockSpec((blk, D_MODEL), lambda i: (i, 0)),
            pl.BlockSpec((D_MODEL, D_MODEL), lambda i: (0, 0)),
        ],
        out_specs=pl.BlockSpec((blk, D_MODEL), lambda i: (i, 0)),
        out_shape=jax.ShapeDtypeStruct((TTOT, D_MODEL), jnp.float32),
    )(x, w_t)


def kernel(x_compact, seqlens_tokens, W_qkv, W_out):
    del seqlens_tokens
    qkv, r = _qkv_r(x_compact, W_qkv.T)
    og, ol = _topk(r)
    order_g = og.reshape(NPACK)
    order_l = ol.reshape(NPACK)
    packed = _gather_sc(qkv, order_g)
    out_p = _attn(packed)
    out_h = _scatter_sc(out_p, order_l)
    return _proj(out_h, W_out.T)


# speedup vs baseline: 3.0875x; 1.3601x over previous
import functools

import jax
import jax.numpy as jnp
import numpy as np
from jax import lax
from jax.experimental import pallas as pl
from jax.experimental.pallas import tpu as pltpu
from jax.experimental.pallas import tpu_sc as plsc

D_MODEL = 1024
N_HEADS = 16
HEAD_DIM = 64
TTOT = 8192
S = 8
SEG_LEN = 1024
W_EFF = 384
K_S = 3
N_CL = S * K_S
NPACK = N_CL * W_EFF
SEED_ROWS = (0, 4096, 8191)


def _qkv_r_body(x_ref, w_ref, qkv_ref, r_ref):
    xb = x_ref[...]
    qkv = jnp.dot(xb, w_ref[...], preferred_element_type=jnp.float32)
    qkv_ref[...] = qkv
    qm = qkv[:, 0:HEAD_DIM]
    km = qkv[:, D_MODEL:D_MODEL + HEAD_DIM]
    for h in range(1, N_HEADS):
        qm = qm + qkv[:, h * HEAD_DIM:(h + 1) * HEAD_DIM]
        km = km + qkv[:, D_MODEL + h * HEAD_DIM:D_MODEL + (h + 1) * HEAD_DIM]
    qm = qm * (1.0 / N_HEADS)
    km = km * (1.0 / N_HEADS)

    def _ln(t):
        m = jnp.mean(t, axis=-1, keepdims=True)
        v = jnp.mean((t - m) ** 2, axis=-1, keepdims=True)
        return (t - m) / jnp.sqrt(v + 1e-5)

    r = 0.5 * (_ln(qm) + _ln(km))
    nrm = jnp.sqrt(jnp.sum(r * r, axis=-1, keepdims=True))
    r_ref[...] = r / (nrm + 1e-6)


def _qkv_r(x, w_t):
    blk = 256
    return pl.pallas_call(
        _qkv_r_body,
        grid=(TTOT // blk,),
        in_specs=[
            pl.BlockSpec((blk, D_MODEL), lambda i: (i, 0)),
            pl.BlockSpec((D_MODEL, 3 * D_MODEL), lambda i: (0, 0)),
        ],
        out_specs=[
            pl.BlockSpec((blk, 3 * D_MODEL), lambda i: (i, 0)),
            pl.BlockSpec((blk, HEAD_DIM), lambda i: (i, 0)),
        ],
        out_shape=[
            jax.ShapeDtypeStruct((TTOT, 3 * D_MODEL), jnp.float32),
            jax.ShapeDtypeStruct((TTOT, HEAD_DIM), jnp.float32),
        ],
    )(x, w_t)


def _topk_body(r_ref, og_ref, ol_ref):
    seg = pl.program_id(0)
    rows = [r_ref[sr:sr + 1, :] for sr in SEED_ROWS]
    cmat = jnp.concatenate(rows, axis=0)
    cn = jnp.sqrt(jnp.sum(cmat * cmat, axis=-1, keepdims=True))
    cmat = cmat / (cn + 1e-6)
    r_seg = r_ref[pl.ds(seg * SEG_LEN, SEG_LEN), :]
    sims = lax.dot_general(cmat, r_seg, (((1,), (1,)), ((), ())),
                           preferred_element_type=jnp.float32)

    ri = lax.broadcasted_iota(jnp.int32, (SEG_LEN, SEG_LEN), 0)
    ci = lax.broadcasted_iota(jnp.int32, (SEG_LEN, SEG_LEN), 1)
    tri = (ri <= ci).astype(jnp.float32)
    tok_row = lax.broadcasted_iota(jnp.int32, (1, SEG_LEN), 1).astype(jnp.float32)
    slot_col = lax.broadcasted_iota(jnp.int32, (W_EFF, 1), 0)

    for c in range(K_S):
        sv = sims[c:c + 1, :]
        b = lax.bitcast_convert_type(sv, jnp.int32)
        key = jnp.where(b < 0, b ^ jnp.int32(0x7FFFFFFF), b)

        def bit_step(i, cur):
            bit = 31 - i
            inc = jnp.left_shift(jnp.int32(1), bit)
            cand = cur + inc
            cnt = jnp.sum((key >= cand).astype(jnp.int32))
            return jnp.where(cnt >= W_EFF, cand, cur)

        t_val = lax.fori_loop(0, 32, bit_step, jnp.int32(-2147483648))

        gt = key > t_val
        eq = key == t_val
        need = W_EFF - jnp.sum(gt.astype(jnp.int32))
        cum_eq = jnp.dot(eq.astype(jnp.float32), tri,
                         preferred_element_type=jnp.float32,
                         precision=lax.Precision.HIGHEST)
        sel = gt | (eq & (cum_eq <= need.astype(jnp.float32) + 0.5))
        self_f = sel.astype(jnp.float32)
        pos = jnp.dot(self_f, tri, preferred_element_type=jnp.float32,
                      precision=lax.Precision.HIGHEST) - 1.0
        p2 = ((slot_col.astype(jnp.float32) == pos) & sel).astype(jnp.float32)
        idx_f = lax.dot_general(tok_row * self_f, p2, (((1,), (1,)), ((), ())),
                                preferred_element_type=jnp.float32,
                                precision=lax.Precision.HIGHEST)
        idx_i = idx_f.astype(jnp.int32)
        ol_ref[0, c, :] = idx_i[0, :]
        og_ref[0, c, :] = idx_i[0, :] + seg * SEG_LEN


def _topk(r):
    return pl.pallas_call(
        _topk_body,
        grid=(S,),
        in_specs=[pl.BlockSpec((TTOT, HEAD_DIM), lambda s: (0, 0))],
        out_specs=[
            pl.BlockSpec((1, K_S, W_EFF), lambda s: (s, 0, 0)),
            pl.BlockSpec((1, K_S, W_EFF), lambda s: (s, 0, 0)),
        ],
        out_shape=[
            jax.ShapeDtypeStruct((S, K_S, W_EFF), jnp.int32),
            jax.ShapeDtypeStruct((S, K_S, W_EFF), jnp.int32),
        ],
    )(r)


def _gather_sc(qkv, order_flat):
    info = plsc.get_sparse_core_info()
    nw = info.num_cores * info.num_subcores
    rows_per_w = NPACK // nw
    chunk = 16
    n_chunks = rows_per_w // chunk
    mesh = plsc.VectorSubcoreMesh(core_axis_name="c", subcore_axis_name="s")

    @functools.partial(
        pl.kernel,
        out_type=jax.ShapeDtypeStruct((NPACK, 3 * D_MODEL), jnp.float32),
        mesh=mesh,
        scratch_types=[
            pltpu.VMEM((rows_per_w,), jnp.int32),
            pltpu.VMEM((2, chunk, 3 * D_MODEL), jnp.float32),
            pltpu.SemaphoreType.DMA,
            pltpu.SemaphoreType.DMA,
            pltpu.SemaphoreType.DMA,
            pltpu.SemaphoreType.DMA,
        ],
    )
    def k(table_hbm, idx_hbm, out_hbm, idx_v, bufs, gs0, gs1, ws0, ws1):
        wid = lax.axis_index("s") * info.num_cores + lax.axis_index("c")
        base = wid * rows_per_w
        pltpu.sync_copy(idx_hbm.at[pl.ds(base, rows_per_w)], idx_v)
        gsems = (gs0, gs1)
        wsems = (ws0, ws1)

        def start_gather(ci, slot):
            pltpu.async_copy(
                table_hbm.at[idx_v.at[pl.ds(ci * chunk, chunk)]],
                bufs.at[slot], gsems[slot])

        start_gather(0, 0)
        start_gather(1, 1)

        def outer(oc, _):
            for b in range(2):
                ci = oc * 2 + b
                pltpu.make_async_copy(
                    table_hbm.at[idx_v.at[pl.ds(0, chunk)]],
                    bufs.at[b], gsems[b]).wait()
                pltpu.async_copy(bufs.at[b],
                                 out_hbm.at[pl.ds(base + ci * chunk, chunk)],
                                 wsems[b])

                @pl.when(ci + 2 < n_chunks)
                def _():
                    pltpu.make_async_copy(
                        bufs.at[b], out_hbm.at[pl.ds(base, chunk)],
                        wsems[b]).wait()
                    start_gather(lax.convert_element_type(ci + 2, jnp.int32), b)

            return 0

        lax.fori_loop(0, n_chunks // 2, outer, 0)
        for b in range(2):
            pltpu.make_async_copy(bufs.at[b], out_hbm.at[pl.ds(base, chunk)],
                                  wsems[b]).wait()

    return k(qkv, order_flat)


def _attn_body(pk_ref, w_ref, out_ref):
    blk = pk_ref[...]
    scale = 1.0 / np.sqrt(HEAD_DIM)
    outs = []
    for h in range(N_HEADS):
        q = blk[:, h * HEAD_DIM:(h + 1) * HEAD_DIM]
        k = blk[:, D_MODEL + h * HEAD_DIM:D_MODEL + (h + 1) * HEAD_DIM]
        v = blk[:, 2 * D_MODEL + h * HEAD_DIM:2 * D_MODEL + (h + 1) * HEAD_DIM]
        s = lax.dot_general(q, k, (((1,), (1,)), ((), ())),
                            preferred_element_type=jnp.float32) * scale
        m = jnp.max(s, axis=-1, keepdims=True)
        p = jnp.exp(s - m)
        l = jnp.sum(p, axis=-1, keepdims=True)
        outs.append(jnp.dot(p / l, v, preferred_element_type=jnp.float32))
    o = jnp.concatenate(outs, axis=-1)
    out_ref[...] = jnp.dot(o, w_ref[...], preferred_element_type=jnp.float32)


def _attn(packed, w_out_t):
    return pl.pallas_call(
        _attn_body,
        grid=(N_CL,),
        in_specs=[
            pl.BlockSpec((W_EFF, 3 * D_MODEL), lambda i: (i, 0)),
            pl.BlockSpec((D_MODEL, D_MODEL), lambda i: (0, 0)),
        ],
        out_specs=pl.BlockSpec((W_EFF, D_MODEL), lambda i: (i, 0)),
        out_shape=jax.ShapeDtypeStruct((NPACK, D_MODEL), jnp.float32),
    )(packed, w_out_t)


def _scatter_sc(out_p, order_local_flat):
    info = plsc.get_sparse_core_info()
    seg_rows_packed = K_S * W_EFF
    segs_per_core = S // info.num_cores
    cols = 128
    half = SEG_LEN // 2
    mesh = plsc.VectorSubcoreMesh(core_axis_name="c", subcore_axis_name="s")

    @functools.partial(
        pl.kernel,
        out_type=jax.ShapeDtypeStruct((TTOT, D_MODEL), jnp.float32),
        mesh=mesh,
        scratch_types=[
            pltpu.VMEM((2, 64, cols), jnp.float32),
            pltpu.VMEM((seg_rows_packed,), jnp.int32),
            pltpu.VMEM((half, cols), jnp.float32),
            pltpu.SemaphoreType.DMA,
            pltpu.SemaphoreType.DMA,
        ],
    )
    def k(src_hbm, idx_hbm, out_hbm, bufs, idx_v, acc, sm0, sm1):
        cid = lax.axis_index("c")
        sid = lax.axis_index("s")
        colb = sid % 8
        rhalf = sid // 8
        rbase = rhalf * half
        zeros16 = jnp.zeros((16,), jnp.float32)
        sems = (sm0, sm1)
        n_ch = seg_rows_packed // 64

        def per_seg(i, _):
            seg = cid * segs_per_core + i

            def zrow(rr, _):
                for j in range(cols // 16):
                    acc[rr, pl.ds(j * 16, 16)] = zeros16
                return 0

            lax.fori_loop(0, half, zrow, 0)
            base = seg * seg_rows_packed
            pltpu.sync_copy(idx_hbm.at[pl.ds(base, seg_rows_packed)], idx_v)

            def start(ci, slot):
                pltpu.async_copy(
                    src_hbm.at[pl.ds(base + ci * 64, 64),
                               pl.ds(colb * cols, cols)],
                    bufs.at[slot], sems[slot])

            start(0, 0)
            start(1, 1)

            def outer(oc, _):
                for b in range(2):
                    ci = oc * 2 + b
                    pltpu.make_async_copy(
                        src_hbm.at[pl.ds(base, 64), pl.ds(0, cols)],
                        bufs.at[b], sems[b]).wait()
                    for g in range(4):
                        rv = idx_v[pl.ds(ci * 64 + g * 16, 16)]
                        for i in range(16):
                            r = rv[i] - rbase

                            @pl.when((r >= 0) & (r < half))
                            def _():
                                for u in range(cols // 16):
                                    plsc.addupdate(
                                        acc.at[r, pl.ds(u * 16, 16)],
                                        bufs[b, g * 16 + i, pl.ds(u * 16, 16)])

                    @pl.when(ci + 2 < n_ch)
                    def _():
                        start(ci + 2, b)

                return 0

            lax.fori_loop(0, n_ch // 2, outer, 0)
            pltpu.sync_copy(
                acc,
                out_hbm.at[pl.ds(seg * SEG_LEN + rbase, half),
                           pl.ds(colb * cols, cols)])
            return 0

        lax.fori_loop(0, segs_per_core, per_seg, 0)

    return k(out_p, order_local_flat)


def _proj_body(x_ref, w_ref, o_ref):
    o_ref[...] = jnp.dot(x_ref[...], w_ref[...],
                         preferred_element_type=jnp.float32)


def _proj(x, w_t):
    blk = 512
    return pl.pallas_call(
        _proj_body,
        grid=(TTOT // blk,),
        in_specs=[
            pl.BlockSpec((blk, D_MODEL), lambda i: (i, 0)),
            pl.BlockSpec((D_MODEL, D_MODEL), lambda i: (0, 0)),
        ],
        out_specs=pl.BlockSpec((blk, D_MODEL), lambda i: (i, 0)),
        out_shape=jax.ShapeDtypeStruct((TTOT, D_MODEL), jnp.float32),
    )(x, w_t)


def kernel(x_compact, seqlens_tokens, W_qkv, W_out):
    del seqlens_tokens
    qkv, r = _qkv_r(x_compact, W_qkv.T)
    og, ol = _topk(r)
    order_g = og.reshape(NPACK)
    order_l = ol.reshape(NPACK)
    packed = _gather_sc(qkv, order_g)
    out_p = _attn(packed, W_out.T)
    return _scatter_sc(out_p, order_l)


# speedup vs baseline: 4.3335x; 1.4036x over previous
import functools

import jax
import jax.numpy as jnp
import numpy as np
from jax import lax
from jax.experimental import pallas as pl
from jax.experimental.pallas import tpu as pltpu
from jax.experimental.pallas import tpu_sc as plsc

D_MODEL = 1024
N_HEADS = 16
HEAD_DIM = 64
TTOT = 8192
S = 8
SEG_LEN = 1024
W_EFF = 384
K_S = 3
N_CL = S * K_S
NPACK = N_CL * W_EFF
SEED_ROWS = (0, 4096, 8191)


def _qkv_r_body(x_ref, w_ref, qkv_ref, r_ref):
    xb = x_ref[...]
    qkv = jnp.dot(xb, w_ref[...], preferred_element_type=jnp.float32)
    qkv_ref[...] = qkv
    qm = qkv[:, 0:HEAD_DIM]
    km = qkv[:, D_MODEL:D_MODEL + HEAD_DIM]
    for h in range(1, N_HEADS):
        qm = qm + qkv[:, h * HEAD_DIM:(h + 1) * HEAD_DIM]
        km = km + qkv[:, D_MODEL + h * HEAD_DIM:D_MODEL + (h + 1) * HEAD_DIM]
    qm = qm * (1.0 / N_HEADS)
    km = km * (1.0 / N_HEADS)

    def _ln(t):
        m = jnp.mean(t, axis=-1, keepdims=True)
        v = jnp.mean((t - m) ** 2, axis=-1, keepdims=True)
        return (t - m) / jnp.sqrt(v + 1e-5)

    r = 0.5 * (_ln(qm) + _ln(km))
    nrm = jnp.sqrt(jnp.sum(r * r, axis=-1, keepdims=True))
    r_ref[...] = r / (nrm + 1e-6)


def _qkv_r(x, w_t):
    blk = 256
    return pl.pallas_call(
        _qkv_r_body,
        grid=(TTOT // blk,),
        in_specs=[
            pl.BlockSpec((blk, D_MODEL), lambda i: (i, 0)),
            pl.BlockSpec((D_MODEL, 3 * D_MODEL), lambda i: (0, 0)),
        ],
        out_specs=[
            pl.BlockSpec((blk, 3 * D_MODEL), lambda i: (i, 0)),
            pl.BlockSpec((blk, HEAD_DIM), lambda i: (i, 0)),
        ],
        out_shape=[
            jax.ShapeDtypeStruct((TTOT, 3 * D_MODEL), jnp.float32),
            jax.ShapeDtypeStruct((TTOT, HEAD_DIM), jnp.float32),
        ],
    )(x, w_t)


def _topk_body(r_ref, og_ref, ol_ref):
    rows = [r_ref[sr:sr + 1, :] for sr in SEED_ROWS]
    cmat = jnp.concatenate(rows, axis=0)
    cn = jnp.sqrt(jnp.sum(cmat * cmat, axis=-1, keepdims=True))
    cmat = cmat / (cn + 1e-6)
    sims = jnp.concatenate(
        [lax.dot_general(cmat, r_ref[pl.ds(s * SEG_LEN, SEG_LEN), :],
                         (((1,), (1,)), ((), ())),
                         preferred_element_type=jnp.float32)
         for s in range(S)], axis=0)

    b = lax.bitcast_convert_type(sims, jnp.int32)
    key = jnp.where(b < 0, b ^ jnp.int32(0x7FFFFFFF), b)

    def bit_step(i, cur):
        bit = 31 - i
        inc = jnp.left_shift(jnp.int32(1), bit)
        cand = cur + inc
        cnt = jnp.sum((key >= cand).astype(jnp.int32), axis=1, keepdims=True)
        return jnp.where(cnt >= W_EFF, cand, cur)

    t_val = lax.fori_loop(0, 32, bit_step,
                          jnp.full((N_CL, 1), -2147483648, jnp.int32))

    gt = key > t_val
    eq = key == t_val
    need = (W_EFF - jnp.sum(gt.astype(jnp.int32), axis=1, keepdims=True)
            ).astype(jnp.float32)
    ri = lax.broadcasted_iota(jnp.int32, (SEG_LEN, SEG_LEN), 0)
    ci = lax.broadcasted_iota(jnp.int32, (SEG_LEN, SEG_LEN), 1)
    tri = (ri <= ci).astype(jnp.float32)
    cum_eq = jnp.dot(eq.astype(jnp.float32), tri,
                     preferred_element_type=jnp.float32,
                     precision=lax.Precision.HIGHEST)
    sel = gt | (eq & (cum_eq <= need + 0.5))
    self_f = sel.astype(jnp.float32)
    pos = jnp.dot(self_f, tri, preferred_element_type=jnp.float32,
                  precision=lax.Precision.HIGHEST) - 1.0
    tok_row = lax.broadcasted_iota(jnp.int32, (1, SEG_LEN), 1).astype(jnp.float32)
    slot_col = lax.broadcasted_iota(jnp.int32, (W_EFF, 1), 0).astype(jnp.float32)
    toksel = tok_row * self_f

    for row in range(N_CL):
        p2 = ((slot_col == pos[row:row + 1, :]) &
              sel[row:row + 1, :]).astype(jnp.float32)
        idx_f = lax.dot_general(toksel[row:row + 1, :], p2,
                                (((1,), (1,)), ((), ())),
                                preferred_element_type=jnp.float32,
                                precision=lax.Precision.HIGHEST)
        idx_i = idx_f.astype(jnp.int32)
        s, c = divmod(row, K_S)
        ol_ref[s, c, :] = idx_i[0, :]
        og_ref[s, c, :] = idx_i[0, :] + s * SEG_LEN


def _topk(r):
    return pl.pallas_call(
        _topk_body,
        out_specs=[
            pl.BlockSpec((S, K_S, W_EFF), lambda: (0, 0, 0)),
            pl.BlockSpec((S, K_S, W_EFF), lambda: (0, 0, 0)),
        ],
        out_shape=[
            jax.ShapeDtypeStruct((S, K_S, W_EFF), jnp.int32),
            jax.ShapeDtypeStruct((S, K_S, W_EFF), jnp.int32),
        ],
    )(r)


def _gather_sc(qkv, order_flat):
    info = plsc.get_sparse_core_info()
    nw = info.num_cores * info.num_subcores
    rows_per_w = NPACK // nw
    chunk = 16
    n_chunks = rows_per_w // chunk
    mesh = plsc.VectorSubcoreMesh(core_axis_name="c", subcore_axis_name="s")

    @functools.partial(
        pl.kernel,
        out_type=jax.ShapeDtypeStruct((NPACK, 3 * D_MODEL), jnp.float32),
        mesh=mesh,
        scratch_types=[
            pltpu.VMEM((rows_per_w,), jnp.int32),
            pltpu.VMEM((2, chunk, 3 * D_MODEL), jnp.float32),
            pltpu.SemaphoreType.DMA,
            pltpu.SemaphoreType.DMA,
            pltpu.SemaphoreType.DMA,
            pltpu.SemaphoreType.DMA,
        ],
    )
    def k(table_hbm, idx_hbm, out_hbm, idx_v, bufs, gs0, gs1, ws0, ws1):
        wid = lax.axis_index("s") * info.num_cores + lax.axis_index("c")
        base = wid * rows_per_w
        pltpu.sync_copy(idx_hbm.at[pl.ds(base, rows_per_w)], idx_v)
        gsems = (gs0, gs1)
        wsems = (ws0, ws1)

        def start_gather(ci, slot):
            pltpu.async_copy(
                table_hbm.at[idx_v.at[pl.ds(ci * chunk, chunk)]],
                bufs.at[slot], gsems[slot])

        start_gather(0, 0)
        start_gather(1, 1)

        def outer(oc, _):
            for b in range(2):
                ci = oc * 2 + b
                pltpu.make_async_copy(
                    table_hbm.at[idx_v.at[pl.ds(0, chunk)]],
                    bufs.at[b], gsems[b]).wait()
                pltpu.async_copy(bufs.at[b],
                                 out_hbm.at[pl.ds(base + ci * chunk, chunk)],
                                 wsems[b])

                @pl.when(ci + 2 < n_chunks)
                def _():
                    pltpu.make_async_copy(
                        bufs.at[b], out_hbm.at[pl.ds(base, chunk)],
                        wsems[b]).wait()
                    start_gather(lax.convert_element_type(ci + 2, jnp.int32), b)

            return 0

        lax.fori_loop(0, n_chunks // 2, outer, 0)
        for b in range(2):
            pltpu.make_async_copy(bufs.at[b], out_hbm.at[pl.ds(base, chunk)],
                                  wsems[b]).wait()

    return k(qkv, order_flat)


def _attn_body(pk_ref, w_ref, out_ref):
    blk = pk_ref[...]
    scale = 1.0 / np.sqrt(HEAD_DIM)
    outs = []
    for h in range(N_HEADS):
        q = blk[:, h * HEAD_DIM:(h + 1) * HEAD_DIM]
        k = blk[:, D_MODEL + h * HEAD_DIM:D_MODEL + (h + 1) * HEAD_DIM]
        v = blk[:, 2 * D_MODEL + h * HEAD_DIM:2 * D_MODEL + (h + 1) * HEAD_DIM]
        s = lax.dot_general(q, k, (((1,), (1,)), ((), ())),
                            preferred_element_type=jnp.float32) * scale
        p = jnp.exp(s)
        l = jnp.sum(p, axis=-1, keepdims=True)
        outs.append(jnp.dot(p, v, preferred_element_type=jnp.float32) / l)
    o = jnp.concatenate(outs, axis=-1)
    out_ref[...] = jnp.dot(o, w_ref[...], preferred_element_type=jnp.float32)


def _attn(packed, w_out_t):
    return pl.pallas_call(
        _attn_body,
        grid=(N_CL,),
        in_specs=[
            pl.BlockSpec((W_EFF, 3 * D_MODEL), lambda i: (i, 0)),
            pl.BlockSpec((D_MODEL, D_MODEL), lambda i: (0, 0)),
        ],
        out_specs=pl.BlockSpec((W_EFF, D_MODEL), lambda i: (i, 0)),
        out_shape=jax.ShapeDtypeStruct((NPACK, D_MODEL), jnp.float32),
    )(packed, w_out_t)


def _scatter_sc(out_p, order_local_flat):
    info = plsc.get_sparse_core_info()
    seg_rows_packed = K_S * W_EFF
    segs_per_core = S // info.num_cores
    cols = 128
    half = SEG_LEN // 2
    mesh = plsc.VectorSubcoreMesh(core_axis_name="c", subcore_axis_name="s")

    @functools.partial(
        pl.kernel,
        out_type=jax.ShapeDtypeStruct((TTOT, D_MODEL), jnp.float32),
        mesh=mesh,
        scratch_types=[
            pltpu.VMEM((2, 64, cols), jnp.float32),
            pltpu.VMEM((seg_rows_packed,), jnp.int32),
            pltpu.VMEM((half, cols), jnp.float32),
            pltpu.SemaphoreType.DMA,
            pltpu.SemaphoreType.DMA,
        ],
    )
    def k(src_hbm, idx_hbm, out_hbm, bufs, idx_v, acc, sm0, sm1):
        cid = lax.axis_index("c")
        sid = lax.axis_index("s")
        colb = sid % 8
        rhalf = sid // 8
        rbase = rhalf * half
        zeros16 = jnp.zeros((16,), jnp.float32)
        sems = (sm0, sm1)
        n_ch = seg_rows_packed // 64

        def per_seg(i, _):
            seg = cid * segs_per_core + i

            def zrow(rr, _):
                for j in range(cols // 16):
                    acc[rr, pl.ds(j * 16, 16)] = zeros16
                return 0

            lax.fori_loop(0, half, zrow, 0)
            base = seg * seg_rows_packed
            pltpu.sync_copy(idx_hbm.at[pl.ds(base, seg_rows_packed)], idx_v)

            def start(ci, slot):
                pltpu.async_copy(
                    src_hbm.at[pl.ds(base + ci * 64, 64),
                               pl.ds(colb * cols, cols)],
                    bufs.at[slot], sems[slot])

            start(0, 0)
            start(1, 1)

            def outer(oc, _):
                for b in range(2):
                    ci = oc * 2 + b
                    pltpu.make_async_copy(
                        src_hbm.at[pl.ds(base, 64), pl.ds(0, cols)],
                        bufs.at[b], sems[b]).wait()
                    for g in range(4):
                        rv = idx_v[pl.ds(ci * 64 + g * 16, 16)]
                        for i in range(16):
                            r = rv[i] - rbase

                            @pl.when((r >= 0) & (r < half))
                            def _():
                                for u in range(cols // 16):
                                    plsc.addupdate(
                                        acc.at[r, pl.ds(u * 16, 16)],
                                        bufs[b, g * 16 + i, pl.ds(u * 16, 16)])

                    @pl.when(ci + 2 < n_ch)
                    def _():
                        start(ci + 2, b)

                return 0

            lax.fori_loop(0, n_ch // 2, outer, 0)
            pltpu.sync_copy(
                acc,
                out_hbm.at[pl.ds(seg * SEG_LEN + rbase, half),
                           pl.ds(colb * cols, cols)])
            return 0

        lax.fori_loop(0, segs_per_core, per_seg, 0)

    return k(out_p, order_local_flat)


def kernel(x_compact, seqlens_tokens, W_qkv, W_out):
    del seqlens_tokens
    qkv, r = _qkv_r(x_compact, W_qkv.T)
    og, ol = _topk(r)
    packed = _gather_sc(qkv, og.reshape(NPACK))
    out_p = _attn(packed, W_out.T)
    return _scatter_sc(out_p, ol.reshape(NPACK))


# speedup vs baseline: 4.8055x; 1.1089x over previous
import functools

import jax
import jax.numpy as jnp
import numpy as np
from jax import lax
from jax.experimental import pallas as pl
from jax.experimental.pallas import tpu as pltpu
from jax.experimental.pallas import tpu_sc as plsc

D_MODEL = 1024
N_HEADS = 16
HEAD_DIM = 64
TTOT = 8192
S = 8
SEG_LEN = 1024
W_EFF = 384
K_S = 3
N_CL = S * K_S
NPACK = N_CL * W_EFF
SEED_ROWS = (0, 4096, 8191)


def _qkv_r_body(x_ref, w_ref, qkv_ref, r_ref):
    xb = x_ref[...]
    qkv = jnp.dot(xb, w_ref[...], preferred_element_type=jnp.float32)
    qkv_ref[...] = qkv
    qm = qkv[:, 0:HEAD_DIM]
    km = qkv[:, D_MODEL:D_MODEL + HEAD_DIM]
    for h in range(1, N_HEADS):
        qm = qm + qkv[:, h * HEAD_DIM:(h + 1) * HEAD_DIM]
        km = km + qkv[:, D_MODEL + h * HEAD_DIM:D_MODEL + (h + 1) * HEAD_DIM]
    qm = qm * (1.0 / N_HEADS)
    km = km * (1.0 / N_HEADS)

    def _ln(t):
        m = jnp.mean(t, axis=-1, keepdims=True)
        v = jnp.mean((t - m) ** 2, axis=-1, keepdims=True)
        return (t - m) / jnp.sqrt(v + 1e-5)

    r = 0.5 * (_ln(qm) + _ln(km))
    nrm = jnp.sqrt(jnp.sum(r * r, axis=-1, keepdims=True))
    r_ref[...] = r / (nrm + 1e-6)


def _qkv_r(x, w_t):
    blk = 256
    return pl.pallas_call(
        _qkv_r_body,
        grid=(TTOT // blk,),
        in_specs=[
            pl.BlockSpec((blk, D_MODEL), lambda i: (i, 0)),
            pl.BlockSpec((D_MODEL, 3 * D_MODEL), lambda i: (0, 0)),
        ],
        out_specs=[
            pl.BlockSpec((blk, 3 * D_MODEL), lambda i: (i, 0)),
            pl.BlockSpec((blk, HEAD_DIM), lambda i: (i, 0)),
        ],
        out_shape=[
            jax.ShapeDtypeStruct((TTOT, 3 * D_MODEL), jnp.float32),
            jax.ShapeDtypeStruct((TTOT, HEAD_DIM), jnp.float32),
        ],
    )(x, w_t)


def _topk_body(r_ref, og_ref, ol_ref):
    rows = [r_ref[sr:sr + 1, :] for sr in SEED_ROWS]
    cmat = jnp.concatenate(rows, axis=0)
    cn = jnp.sqrt(jnp.sum(cmat * cmat, axis=-1, keepdims=True))
    cmat = cmat / (cn + 1e-6)
    sims = jnp.concatenate(
        [lax.dot_general(cmat, r_ref[pl.ds(s * SEG_LEN, SEG_LEN), :],
                         (((1,), (1,)), ((), ())),
                         preferred_element_type=jnp.float32)
         for s in range(S)], axis=0)

    b = lax.bitcast_convert_type(sims, jnp.int32)
    key = jnp.where(b < 0, b ^ jnp.int32(0x7FFFFFFF), b)

    def bit_step(i, cur):
        bit = 31 - i
        inc = jnp.left_shift(jnp.int32(1), bit)
        cand = cur + inc
        cnt = jnp.sum((key >= cand).astype(jnp.int32), axis=1, keepdims=True)
        return jnp.where(cnt >= W_EFF, cand, cur)

    t_val = lax.fori_loop(0, 32, bit_step,
                          jnp.full((N_CL, 1), -2147483648, jnp.int32))

    gt = key > t_val
    eq = key == t_val
    need = (W_EFF - jnp.sum(gt.astype(jnp.int32), axis=1, keepdims=True)
            ).astype(jnp.float32)
    ri = lax.broadcasted_iota(jnp.int32, (SEG_LEN, SEG_LEN), 0)
    ci = lax.broadcasted_iota(jnp.int32, (SEG_LEN, SEG_LEN), 1)
    tri = (ri <= ci).astype(jnp.float32)
    cum_eq = jnp.dot(eq.astype(jnp.float32), tri,
                     preferred_element_type=jnp.float32,
                     precision=lax.Precision.HIGHEST)
    sel = gt | (eq & (cum_eq <= need + 0.5))
    self_f = sel.astype(jnp.float32)
    pos = jnp.dot(self_f, tri, preferred_element_type=jnp.float32,
                  precision=lax.Precision.HIGHEST) - 1.0
    tok_row = lax.broadcasted_iota(jnp.int32, (1, SEG_LEN), 1).astype(jnp.float32)
    slot_col = lax.broadcasted_iota(jnp.int32, (W_EFF, 1), 0).astype(jnp.float32)
    toksel = tok_row * self_f

    for row in range(N_CL):
        p2 = ((slot_col == pos[row:row + 1, :]) &
              sel[row:row + 1, :]).astype(jnp.float32)
        idx_f = lax.dot_general(toksel[row:row + 1, :], p2,
                                (((1,), (1,)), ((), ())),
                                preferred_element_type=jnp.float32,
                                precision=lax.Precision.HIGHEST)
        idx_i = idx_f.astype(jnp.int32)
        s, c = divmod(row, K_S)
        ol_ref[s, c, :] = idx_i[0, :]
        og_ref[s, c, :] = idx_i[0, :] + s * SEG_LEN


def _topk(r):
    return pl.pallas_call(
        _topk_body,
        out_specs=[
            pl.BlockSpec((S, K_S, W_EFF), lambda: (0, 0, 0)),
            pl.BlockSpec((S, K_S, W_EFF), lambda: (0, 0, 0)),
        ],
        out_shape=[
            jax.ShapeDtypeStruct((S, K_S, W_EFF), jnp.int32),
            jax.ShapeDtypeStruct((S, K_S, W_EFF), jnp.int32),
        ],
    )(r)


def _gather_sc(qkv, order_flat, n_rows):
    info = plsc.get_sparse_core_info()
    nw = info.num_cores * info.num_subcores
    rows_per_w = n_rows // nw
    chunk = 16
    n_chunks = rows_per_w // chunk
    mesh = plsc.VectorSubcoreMesh(core_axis_name="c", subcore_axis_name="s")

    @functools.partial(
        pl.kernel,
        out_type=jax.ShapeDtypeStruct((n_rows, 3 * D_MODEL), jnp.float32),
        mesh=mesh,
        scratch_types=[
            pltpu.VMEM((rows_per_w,), jnp.int32),
            pltpu.VMEM((2, chunk, 3 * D_MODEL), jnp.float32),
            pltpu.SemaphoreType.DMA,
            pltpu.SemaphoreType.DMA,
            pltpu.SemaphoreType.DMA,
            pltpu.SemaphoreType.DMA,
        ],
    )
    def k(table_hbm, idx_hbm, out_hbm, idx_v, bufs, gs0, gs1, ws0, ws1):
        wid = lax.axis_index("s") * info.num_cores + lax.axis_index("c")
        base = wid * rows_per_w
        pltpu.sync_copy(idx_hbm.at[pl.ds(base, rows_per_w)], idx_v)
        gsems = (gs0, gs1)
        wsems = (ws0, ws1)

        def start_gather(ci, slot):
            pltpu.async_copy(
                table_hbm.at[idx_v.at[pl.ds(ci * chunk, chunk)]],
                bufs.at[slot], gsems[slot])

        def wait_gather(slot):
            pltpu.make_async_copy(
                table_hbm.at[idx_v.at[pl.ds(0, chunk)]],
                bufs.at[slot], gsems[slot]).wait()

        def wait_write(slot):
            pltpu.make_async_copy(bufs.at[slot],
                                  out_hbm.at[pl.ds(base, chunk)],
                                  wsems[slot]).wait()

        start_gather(0, 0)
        if n_chunks > 1:
            start_gather(1, 1)

        def outer(oc, _):
            for b in range(2):
                ci = oc * 2 + b
                wait_gather(b)
                pltpu.async_copy(bufs.at[b],
                                 out_hbm.at[pl.ds(base + ci * chunk, chunk)],
                                 wsems[b])

                @pl.when(ci + 2 < n_chunks)
                def _():
                    wait_write(b)
                    start_gather(ci + 2, b)

            return 0

        lax.fori_loop(0, n_chunks // 2, outer, 0)
        if n_chunks % 2:
            ci = n_chunks - 1
            wait_gather(ci % 2)
            pltpu.async_copy(bufs.at[ci % 2],
                             out_hbm.at[pl.ds(base + ci * chunk, chunk)],
                             wsems[ci % 2])
        for b in range(min(2, n_chunks)):
            wait_write(b)

    return k(qkv, order_flat)


def _attn_body(pk_ref, w_ref, out_ref):
    blk = pk_ref[...]
    scale = 1.0 / np.sqrt(HEAD_DIM)
    outs = []
    for h in range(N_HEADS):
        q = blk[:, h * HEAD_DIM:(h + 1) * HEAD_DIM]
        k = blk[:, D_MODEL + h * HEAD_DIM:D_MODEL + (h + 1) * HEAD_DIM]
        v = blk[:, 2 * D_MODEL + h * HEAD_DIM:2 * D_MODEL + (h + 1) * HEAD_DIM]
        s = lax.dot_general(q, k, (((1,), (1,)), ((), ())),
                            preferred_element_type=jnp.float32) * scale
        p = jnp.exp(s)
        l = jnp.sum(p, axis=-1, keepdims=True)
        outs.append(jnp.dot(p, v, preferred_element_type=jnp.float32) / l)
    o = jnp.concatenate(outs, axis=-1)
    out_ref[...] = jnp.dot(o, w_ref[...], preferred_element_type=jnp.float32)


def _attn(packed, w_out_t):
    return pl.pallas_call(
        _attn_body,
        grid=(packed.shape[0] // W_EFF,),
        in_specs=[
            pl.BlockSpec((W_EFF, 3 * D_MODEL), lambda i: (i, 0)),
            pl.BlockSpec((D_MODEL, D_MODEL), lambda i: (0, 0)),
        ],
        out_specs=pl.BlockSpec((W_EFF, D_MODEL), lambda i: (i, 0)),
        out_shape=jax.ShapeDtypeStruct((packed.shape[0], D_MODEL), jnp.float32),
    )(packed, w_out_t)


def _scatter_sc(out_p, order_local_flat, n_segs):
    info = plsc.get_sparse_core_info()
    seg_rows_packed = K_S * W_EFF
    segs_per_core = n_segs // info.num_cores
    cols = 128
    half = SEG_LEN // 2
    mesh = plsc.VectorSubcoreMesh(core_axis_name="c", subcore_axis_name="s")

    @functools.partial(
        pl.kernel,
        out_type=jax.ShapeDtypeStruct((n_segs * SEG_LEN, D_MODEL), jnp.float32),
        mesh=mesh,
        scratch_types=[
            pltpu.VMEM((2, 64, cols), jnp.float32),
            pltpu.VMEM((seg_rows_packed,), jnp.int32),
            pltpu.VMEM((half, cols), jnp.float32),
            pltpu.SemaphoreType.DMA,
            pltpu.SemaphoreType.DMA,
        ],
    )
    def k(src_hbm, idx_hbm, out_hbm, bufs, idx_v, acc, sm0, sm1):
        cid = lax.axis_index("c")
        sid = lax.axis_index("s")
        colb = sid % 8
        rhalf = sid // 8
        rbase = rhalf * half
        zeros16 = jnp.zeros((16,), jnp.float32)
        sems = (sm0, sm1)
        n_ch = seg_rows_packed // 64

        def per_seg(i, _):
            seg = cid * segs_per_core + i

            def zrow(rr, _):
                for j in range(cols // 16):
                    acc[rr, pl.ds(j * 16, 16)] = zeros16
                return 0

            lax.fori_loop(0, half, zrow, 0)
            base = seg * seg_rows_packed
            pltpu.sync_copy(idx_hbm.at[pl.ds(base, seg_rows_packed)], idx_v)

            def start(ci, slot):
                pltpu.async_copy(
                    src_hbm.at[pl.ds(base + ci * 64, 64),
                               pl.ds(colb * cols, cols)],
                    bufs.at[slot], sems[slot])

            start(0, 0)
            start(1, 1)

            def outer(oc, _):
                for b in range(2):
                    ci = oc * 2 + b
                    pltpu.make_async_copy(
                        src_hbm.at[pl.ds(base, 64), pl.ds(0, cols)],
                        bufs.at[b], sems[b]).wait()
                    for g in range(4):
                        rv = idx_v[pl.ds(ci * 64 + g * 16, 16)]
                        for i in range(16):
                            r = rv[i] - rbase

                            @pl.when((r >= 0) & (r < half))
                            def _():
                                for u in range(cols // 16):
                                    plsc.addupdate(
                                        acc.at[r, pl.ds(u * 16, 16)],
                                        bufs[b, g * 16 + i, pl.ds(u * 16, 16)])

                    @pl.when(ci + 2 < n_ch)
                    def _():
                        start(ci + 2, b)

                return 0

            lax.fori_loop(0, n_ch // 2, outer, 0)
            pltpu.sync_copy(
                acc,
                out_hbm.at[pl.ds(seg * SEG_LEN + rbase, half),
                           pl.ds(colb * cols, cols)])
            return 0

        lax.fori_loop(0, segs_per_core, per_seg, 0)

    return k(out_p, order_local_flat)


def kernel(x_compact, seqlens_tokens, W_qkv, W_out):
    del seqlens_tokens
    qkv, r = _qkv_r(x_compact, W_qkv.T)
    og, ol = _topk(r)
    ogf = og.reshape(NPACK)
    olf = ol.reshape(NPACK)
    w_t = W_out.T
    hn = NPACK // 2
    packed_a = _gather_sc(qkv, ogf[:hn], hn)
    out_a = _attn(packed_a, w_t)
    packed_b = _gather_sc(qkv, ogf[hn:], hn)
    oh_a = _scatter_sc(out_a, olf[:hn], S // 2)
    out_b = _attn(packed_b, w_t)
    oh_b = _scatter_sc(out_b, olf[hn:], S // 2)
    return jnp.concatenate([oh_a, oh_b], axis=0)
